# Initial kernel scaffold; baseline (speedup 1.0000x reference)
import jax
import jax.numpy as jnp
from jax.experimental import pallas as pl


def kernel(x, table, W1, b1, W2, b2):
    raise NotImplementedError("write your pallas kernel here")


# speedup vs baseline: 2.1983x; 2.1983x over previous
import jax
import jax.numpy as jnp
from jax import lax
from jax.experimental import pallas as pl
from jax.experimental.pallas import tpu as pltpu
from jax.experimental.pallas import tpu_sc as plsc

B, L = 4096, 200
D = 32
HALF = 100
LANES = 16


def _make_pool_body(nc, ns):
    n_ex = B // (nc * ns)
    n_pairs = n_ex // 2
    inv_l = jnp.float32(1.0 / L)

    def body(x_hbm, table_hbm, out_hbm, idx_v, rows_v, pooled_v, sem0, sem1):
        wid = lax.axis_index("s") * nc + lax.axis_index("c")

        rows_per_w = 2 * n_ex
        pltpu.sync_copy(x_hbm.at[pl.ds(wid * rows_per_w, rows_per_w)], idx_v)

        def fire(local_ex, buf, sem):
            for h in range(2):
                pltpu.async_copy(
                    table_hbm.at[idx_v.at[2 * local_ex + h]],
                    rows_v.at[buf, pl.ds(h * HALF, HALF)],
                    sem,
                )

        def wait_buf(buf, sem):
            pltpu.make_async_copy(
                table_hbm.at[pl.ds(0, L)], rows_v.at[buf], sem
            ).wait()

        def reduce_buf(buf, local_ex):
            accs = [jnp.zeros((LANES,), jnp.float32) for _ in range(8)]
            for i in range(L):
                g = i % 4
                accs[2 * g] = accs[2 * g] + rows_v[buf, i, pl.ds(0, LANES)]
                accs[2 * g + 1] = (
                    accs[2 * g + 1] + rows_v[buf, i, pl.ds(LANES, LANES)]
                )
            a0 = (accs[0] + accs[2]) + (accs[4] + accs[6])
            a1 = (accs[1] + accs[3]) + (accs[5] + accs[7])
            pooled_v[local_ex, pl.ds(0, LANES)] = a0 * inv_l
            pooled_v[local_ex, pl.ds(LANES, LANES)] = a1 * inv_l

        fire(0, 0, sem0)

        def pair_body(p, carry):
            e0 = 2 * p
            fire(e0 + 1, 1, sem1)
            wait_buf(0, sem0)
            reduce_buf(0, e0)

            @pl.when(p < n_pairs - 1)
            def _prefetch():
                fire(e0 + 2, 0, sem0)

            wait_buf(1, sem1)
            reduce_buf(1, e0 + 1)
            return carry

        lax.fori_loop(0, n_pairs, pair_body, 0)

        pltpu.sync_copy(pooled_v, out_hbm.at[pl.ds(wid * n_ex, n_ex)])

    return body, n_ex


def _pooled_sc(x2d, table):
    info = plsc.get_sparse_core_info()
    body, n_ex = _make_pool_body(info.num_cores, info.num_subcores)
    mesh = plsc.VectorSubcoreMesh(core_axis_name="c", subcore_axis_name="s")
    return pl.kernel(
        body,
        out_type=jax.ShapeDtypeStruct((B, D), jnp.float32),
        mesh=mesh,
        scratch_types=[
            pltpu.VMEM((2 * n_ex, HALF), jnp.int32),
            pltpu.VMEM((2, L, D), jnp.float32),
            pltpu.VMEM((n_ex, D), jnp.float32),
            pltpu.SemaphoreType.DMA,
            pltpu.SemaphoreType.DMA,
        ],
        compiler_params=pltpu.CompilerParams(use_tc_tiling_on_sc=False),
    )(x2d, table)


def _mlp_body(x_ref, w1t_ref, b1_ref, w2t_ref, b2_ref, o_ref):
    h = jnp.dot(x_ref[...], w1t_ref[...], preferred_element_type=jnp.float32)
    h = jnp.maximum(h + b1_ref[...], 0.0)
    o = jnp.dot(h, w2t_ref[...], preferred_element_type=jnp.float32)
    o_ref[...] = o + b2_ref[...]


def kernel(x, table, W1, b1, W2, b2):
    x2d = x.astype(jnp.int32).reshape(B * L // HALF, HALF)
    pooled = _pooled_sc(x2d, table)
    return pl.pallas_call(
        _mlp_body,
        out_shape=jax.ShapeDtypeStruct((B, W2.shape[0]), jnp.float32),
    )(pooled, W1.T, b1.reshape(1, -1), W2.T, b2.reshape(1, -1))


# speedup vs baseline: 2.5777x; 1.1726x over previous
import jax
import jax.numpy as jnp
from jax import lax
from jax.experimental import pallas as pl
from jax.experimental.pallas import tpu as pltpu
from jax.experimental.pallas import tpu_sc as plsc

B, L = 4096, 200
D = 32
HALF = 100
LANES = 16


def _make_pool_body(nc, ns):
    n_ex = B // (nc * ns)
    n_pairs = n_ex // 2
    inv_l = jnp.float32(1.0 / L)

    def body(x_hbm, table_hbm, out_hbm, idx_v, rows_v, pooled_v, sem0, sem1):
        wid = lax.axis_index("s") * nc + lax.axis_index("c")

        rows_per_w = 2 * n_ex
        pltpu.sync_copy(x_hbm.at[pl.ds(wid * rows_per_w, rows_per_w)], idx_v)

        def fire(local_ex, buf, sem):
            for h in range(2):
                pltpu.async_copy(
                    table_hbm.at[idx_v.at[2 * local_ex + h]],
                    rows_v.at[buf, pl.ds(h * HALF, HALF)],
                    sem,
                )

        def wait_buf(buf, sem):
            pltpu.make_async_copy(
                table_hbm.at[pl.ds(0, L)], rows_v.at[buf], sem
            ).wait()

        def reduce_buf(buf, local_ex):
            accs = [jnp.zeros((LANES,), jnp.float32) for _ in range(8)]
            for i in range(L):
                g = i % 4
                accs[2 * g] = accs[2 * g] + rows_v[buf, i, pl.ds(0, LANES)]
                accs[2 * g + 1] = (
                    accs[2 * g + 1] + rows_v[buf, i, pl.ds(LANES, LANES)]
                )
            a0 = (accs[0] + accs[2]) + (accs[4] + accs[6])
            a1 = (accs[1] + accs[3]) + (accs[5] + accs[7])
            pooled_v[local_ex, pl.ds(0, LANES)] = a0 * inv_l
            pooled_v[local_ex, pl.ds(LANES, LANES)] = a1 * inv_l

        fire(0, 0, sem0)

        def pair_body(p, carry):
            e0 = 2 * p
            fire(e0 + 1, 1, sem1)
            wait_buf(0, sem0)
            reduce_buf(0, e0)

            @pl.when(p < n_pairs - 1)
            def _prefetch():
                fire(e0 + 2, 0, sem0)

            wait_buf(1, sem1)
            reduce_buf(1, e0 + 1)
            return carry

        lax.fori_loop(0, n_pairs, pair_body, 0)

        pltpu.sync_copy(pooled_v, out_hbm.at[pl.ds(wid * n_ex, n_ex)])

    return body, n_ex


def _pooled_sc(x2d, table):
    info = plsc.get_sparse_core_info()
    body, n_ex = _make_pool_body(info.num_cores, info.num_subcores)
    mesh = plsc.VectorSubcoreMesh(core_axis_name="c", subcore_axis_name="s")
    return pl.kernel(
        body,
        out_type=jax.ShapeDtypeStruct((B, D), jnp.float32),
        mesh=mesh,
        scratch_types=[
            pltpu.VMEM((2 * n_ex, HALF), jnp.int32),
            pltpu.VMEM((2, L, D), jnp.float32),
            pltpu.VMEM((n_ex, D), jnp.float32),
            pltpu.SemaphoreType.DMA,
            pltpu.SemaphoreType.DMA,
        ],
        compiler_params=pltpu.CompilerParams(use_tc_tiling_on_sc=False),
    )(x2d, table)


CB = 8192
GT = (1000000 + CB - 1) // CB


def _tr_body(t_ref, o_ref):
    tt = t_ref[...].T.reshape(CB // 4, 4, D)
    o_ref[...] = jnp.concatenate([tt[:, k, :] for k in range(4)], axis=1)


def _row_major_table(table):
    packed = pl.pallas_call(
        _tr_body,
        grid=(GT,),
        in_specs=[pl.BlockSpec((D, CB), lambda c: (0, c))],
        out_specs=pl.BlockSpec((CB * D // 128, 128), lambda c: (c, 0)),
        out_shape=jax.ShapeDtypeStruct((GT * CB * D // 128, 128), jnp.float32),
    )(table.T)
    return packed.reshape(GT * CB, D)


def _mlp_body(x_ref, w1t_ref, b1_ref, w2t_ref, b2_ref, o_ref):
    h = jnp.dot(x_ref[...], w1t_ref[...], preferred_element_type=jnp.float32)
    h = jnp.maximum(h + b1_ref[...], 0.0)
    o = jnp.dot(h, w2t_ref[...], preferred_element_type=jnp.float32)
    o_ref[...] = o + b2_ref[...]


def kernel(x, table, W1, b1, W2, b2):
    x2d = x.astype(jnp.int32).reshape(B * L // HALF, HALF)
    pooled = _pooled_sc(x2d, _row_major_table(table))
    return pl.pallas_call(
        _mlp_body,
        out_shape=jax.ShapeDtypeStruct((B, W2.shape[0]), jnp.float32),
    )(pooled, W1.T, b1.reshape(1, -1), W2.T, b2.reshape(1, -1))


# speedup vs baseline: 4.8593x; 1.8851x over previous
import jax
import jax.numpy as jnp
from jax import lax
from jax.experimental import pallas as pl
from jax.experimental.pallas import tpu as pltpu
from jax.experimental.pallas import tpu_sc as plsc

B, L = 4096, 200
D = 32
HALF = 100
LANES = 16


def _make_pool_body(nc, ns):
    n_ex = B // (nc * ns)
    n_pairs = n_ex // 2
    inv_l = jnp.float32(1.0 / L)

    n_idx = n_ex * L

    def body(x_hbm, table_hbm, out_hbm, idx_v, rows_v, pooled_v, sem0, sem1):
        wid = lax.axis_index("s") * nc + lax.axis_index("c")

        pltpu.sync_copy(
            x_hbm.at[pl.ds(pl.multiple_of(wid * n_idx, 8), n_idx)], idx_v
        )

        def remap_chunk(i, carry):
            off = pl.multiple_of(i * 16, 16)
            r = idx_v[pl.ds(off, 16)]
            idx_v[pl.ds(off, 16)] = (
                (r & jnp.int32(-8192))
                + ((r & jnp.int32(2047)) << 2)
                + ((r >> 11) & jnp.int32(3))
            )
            return carry

        lax.fori_loop(0, n_idx // 16, remap_chunk, 0)

        def fire(local_ex, buf, sem):
            for off, sz in ((0, 104), (104, 96)):
                pltpu.async_copy(
                    table_hbm.at[
                        idx_v.at[pl.ds(pl.multiple_of(local_ex * L + off, 8), sz)]
                    ],
                    rows_v.at[buf, pl.ds(off, sz)],
                    sem,
                )

        def wait_buf(buf, sem):
            pltpu.make_async_copy(
                table_hbm.at[pl.ds(0, L)], rows_v.at[buf], sem
            ).wait()

        def reduce_buf(buf, local_ex):
            accs = [jnp.zeros((LANES,), jnp.float32) for _ in range(8)]
            for i in range(L):
                g = i % 4
                accs[2 * g] = accs[2 * g] + rows_v[buf, i, pl.ds(0, LANES)]
                accs[2 * g + 1] = (
                    accs[2 * g + 1] + rows_v[buf, i, pl.ds(LANES, LANES)]
                )
            a0 = (accs[0] + accs[2]) + (accs[4] + accs[6])
            a1 = (accs[1] + accs[3]) + (accs[5] + accs[7])
            pooled_v[local_ex, pl.ds(0, LANES)] = a0 * inv_l
            pooled_v[local_ex, pl.ds(LANES, LANES)] = a1 * inv_l

        fire(0, 0, sem0)

        def pair_body(p, carry):
            e0 = 2 * p
            fire(e0 + 1, 1, sem1)
            wait_buf(0, sem0)
            reduce_buf(0, e0)

            @pl.when(p < n_pairs - 1)
            def _prefetch():
                fire(e0 + 2, 0, sem0)

            wait_buf(1, sem1)
            reduce_buf(1, e0 + 1)
            return carry

        lax.fori_loop(0, n_pairs, pair_body, 0)

        pltpu.sync_copy(pooled_v, out_hbm.at[pl.ds(wid * n_ex, n_ex)])

    return body, n_ex


def _pooled_sc(x1d, table):
    info = plsc.get_sparse_core_info()
    body, n_ex = _make_pool_body(info.num_cores, info.num_subcores)
    mesh = plsc.VectorSubcoreMesh(core_axis_name="c", subcore_axis_name="s")
    return pl.kernel(
        body,
        out_type=jax.ShapeDtypeStruct((B, D), jnp.float32),
        mesh=mesh,
        scratch_types=[
            pltpu.VMEM((n_ex * L,), jnp.int32),
            pltpu.VMEM((2, L, D), jnp.float32),
            pltpu.VMEM((n_ex, D), jnp.float32),
            pltpu.SemaphoreType.DMA,
            pltpu.SemaphoreType.DMA,
        ],
        compiler_params=pltpu.CompilerParams(use_tc_tiling_on_sc=False),
    )(x1d, table)


CB = 8192
GT = (1000000 + CB - 1) // CB


def _tr_body(t_ref, o_ref):
    t = t_ref[...]
    q = CB // 4
    stacked = jnp.concatenate([t[:, a * q:(a + 1) * q] for a in range(4)], axis=0)
    o_ref[...] = stacked.T


def _row_major_table(table):
    packed = pl.pallas_call(
        _tr_body,
        grid=(GT,),
        in_specs=[pl.BlockSpec((D, CB), lambda c: (0, c))],
        out_specs=pl.BlockSpec((CB * D // 128, 128), lambda c: (c, 0)),
        out_shape=jax.ShapeDtypeStruct((GT * CB * D // 128, 128), jnp.float32),
    )(table.T)
    return packed.reshape(GT * CB, D)


def _mlp_body(x_ref, w1t_ref, b1_ref, w2t_ref, b2_ref, o_ref):
    h = jnp.dot(x_ref[...], w1t_ref[...], preferred_element_type=jnp.float32)
    h = jnp.maximum(h + b1_ref[...], 0.0)
    o = jnp.dot(h, w2t_ref[...], preferred_element_type=jnp.float32)
    o_ref[...] = o + b2_ref[...]


def kernel(x, table, W1, b1, W2, b2):
    x1d = x.astype(jnp.int32).reshape(B * L)
    pooled = _pooled_sc(x1d, _row_major_table(table))
    return pl.pallas_call(
        _mlp_body,
        out_shape=jax.ShapeDtypeStruct((B, W2.shape[0]), jnp.float32),
    )(pooled, W1.T, b1.reshape(1, -1), W2.T, b2.reshape(1, -1))
